# baseline (speedup 1.0000x reference)
import functools

import jax
import jax.numpy as jnp
from jax import lax
from jax.experimental import pallas as pl
from jax.experimental.pallas import tpu as pltpu
from jax.experimental.pallas import tpu_sc as plsc

_B = 1024
_W = 100
_D = 300
_V = 100000
_NB = 16


def _sc_gather(table, idx):
    info = plsc.get_sparse_core_info()
    mesh = plsc.ScalarSubcoreMesh(axis_name="c")
    per = _B // info.num_cores
    chunk_rows = 32

    @functools.partial(
        pl.kernel,
        mesh=mesh,
        out_type=jax.ShapeDtypeStruct((_B, _D), jnp.float32),
        scratch_types=[
            pltpu.SMEM((per,), jnp.int32),
            pltpu.SemaphoreType.DMA,
        ],
    )
    def gather_kernel(table_hbm, idx_hbm, out_hbm, idx_s, sem):
        base = lax.axis_index("c") * per
        pltpu.sync_copy(idx_hbm.at[pl.ds(base, per)], idx_s)

        def chunk(c, carry):
            def fire(j, carry2):
                r = c * chunk_rows + j
                pltpu.async_copy(table_hbm.at[idx_s[r]], out_hbm.at[base + r], sem)
                return carry2

            lax.fori_loop(0, chunk_rows, fire, 0)

            def drain(j, carry2):
                r = c * chunk_rows + j
                pltpu.make_async_copy(
                    table_hbm.at[idx_s[r]], out_hbm.at[base + r], sem
                ).wait()
                return carry2

            lax.fori_loop(0, chunk_rows, drain, 0)
            return carry

        lax.fori_loop(0, per // chunk_rows, chunk, 0)

    return gather_kernel(table, idx)


def _tc_body(c_ref, g_ref, o_ref):
    c = c_ref[...]
    g = g_ref[...]
    s = jnp.sum(c * g[:, None, :], axis=-1)
    n2 = jnp.sum(c * c, axis=-1)
    o_ref[...] = s / jnp.maximum(jnp.sqrt(n2), 1e-12)


def kernel(ctxt_word_vecs, ent_idxes, ent_embeddings):
    g = _sc_gather(ent_embeddings, ent_idxes)
    ctxt3 = ctxt_word_vecs.reshape(_B, _W, _D)
    out = pl.pallas_call(
        _tc_body,
        grid=(_B // _NB,),
        in_specs=[
            pl.BlockSpec((_NB, _W, _D), lambda i: (i, 0, 0)),
            pl.BlockSpec((_NB, _D), lambda i: (i, 0)),
        ],
        out_specs=pl.BlockSpec((_NB, _W), lambda i: (i, 0)),
        out_shape=jax.ShapeDtypeStruct((_B, _W), jnp.float32),
    )(ctxt3, g)
    return out.reshape(_B * 20, 5)


# speedup vs baseline: 2.6708x; 2.6708x over previous
import functools

import jax
import jax.numpy as jnp
from jax import lax
from jax.experimental import pallas as pl
from jax.experimental.pallas import tpu as pltpu
from jax.experimental.pallas import tpu_sc as plsc

_B = 1024
_W = 100
_D = 300
_V = 100000
_NB = 32


def _sc_gather(table, idx):
    info = plsc.get_sparse_core_info()
    mesh = plsc.ScalarSubcoreMesh(axis_name="c")
    per = _B // info.num_cores
    chunk_rows = 32

    @functools.partial(
        pl.kernel,
        mesh=mesh,
        out_type=jax.ShapeDtypeStruct((_B, _D), jnp.float32),
        scratch_types=[
            pltpu.SMEM((per,), jnp.int32),
            pltpu.SemaphoreType.DMA,
        ],
    )
    def gather_kernel(table_hbm, idx_hbm, out_hbm, idx_s, sem):
        base = lax.axis_index("c") * per
        pltpu.sync_copy(idx_hbm.at[pl.ds(base, per)], idx_s)

        def chunk(c, carry):
            def fire(j, carry2):
                r = c * chunk_rows + j
                pltpu.async_copy(table_hbm.at[idx_s[r]], out_hbm.at[base + r], sem)
                return carry2

            lax.fori_loop(0, chunk_rows, fire, 0)

            def drain(j, carry2):
                r = c * chunk_rows + j
                pltpu.make_async_copy(
                    table_hbm.at[idx_s[r]], out_hbm.at[base + r], sem
                ).wait()
                return carry2

            lax.fori_loop(0, chunk_rows, drain, 0)
            return carry

        lax.fori_loop(0, per // chunk_rows, chunk, 0)

    return gather_kernel(table, idx)


_CB = 3200


def _tc_body(x_ref, g_ref, o_ref):
    c = x_ref[...]
    gb = g_ref[...]
    s_all = jax.lax.dot_general(
        gb, c, (((1,), (0,)), ((), ())),
        preferred_element_type=jnp.float32,
    )
    row = jax.lax.broadcasted_iota(jnp.int32, (_NB, _CB), 0)
    col = jax.lax.broadcasted_iota(jnp.int32, (_NB, _CB), 1)
    d = col - row * _W
    mask = (d >= 0) & (d < _W)
    s = jnp.sum(jnp.where(mask, s_all, 0.0), axis=0)
    n2 = jnp.sum(c * c, axis=0)
    o_ref[...] = (s / jnp.maximum(jnp.sqrt(n2), 1e-12))[None, :]


def kernel(ctxt_word_vecs, ent_idxes, ent_embeddings):
    g = _sc_gather(ent_embeddings, ent_idxes)
    xt = ctxt_word_vecs.T
    out = pl.pallas_call(
        _tc_body,
        grid=(_B * _W // _CB,),
        in_specs=[
            pl.BlockSpec((_D, _CB), lambda i: (0, i)),
            pl.BlockSpec((_NB, _D), lambda i: (i, 0)),
        ],
        out_specs=pl.BlockSpec((1, _CB), lambda i: (0, i)),
        out_shape=jax.ShapeDtypeStruct((1, _B * _W), jnp.float32),
    )(xt, g)
    return out.reshape(_B * 20, 5)


# speedup vs baseline: 2.9626x; 1.1093x over previous
import functools

import jax
import jax.numpy as jnp
from jax import lax
from jax.experimental import pallas as pl
from jax.experimental.pallas import tpu as pltpu
from jax.experimental.pallas import tpu_sc as plsc

_B = 1024
_W = 100
_D = 300
_V = 100000
_NB = 64
_CB = _NB * _W


def _sc_gather_groups(table, idx):
    info = plsc.get_sparse_core_info()
    nc, ns, nl = info.num_cores, info.num_subcores, info.num_lanes
    nw = nc * ns
    bpw = _B // nw
    mesh = plsc.VectorSubcoreMesh(core_axis_name="c", subcore_axis_name="s")

    @functools.partial(
        pl.kernel,
        mesh=mesh,
        out_type=jax.ShapeDtypeStruct((_B * 8, _D), jnp.bfloat16),
        scratch_types=[
            pltpu.VMEM((bpw,), jnp.int32),
            pltpu.VMEM((bpw * 8, _D), jnp.bfloat16),
            pltpu.SemaphoreType.DMA,
        ],
        compiler_params=pltpu.CompilerParams(needs_layout_passes=False),
    )
    def gather_kernel(table_hbm, idx_hbm, out_hbm, idx_v, groups_v, sem):
        wid = lax.axis_index("s") * nc + lax.axis_index("c")
        base = wid * bpw
        pltpu.sync_copy(idx_hbm.at[pl.ds(base, bpw)], idx_v)

        lane = lax.iota(jnp.int32, nl)
        neg = jnp.full((nl,), -1, jnp.int32)
        for v in range(bpw // nl):
            vec = idx_v[pl.ds(v * nl, nl)]
            for j in range(nl):
                i = lax.reduce_max(jnp.where(lane == j, vec, neg), axes=(0,))
                ib = (i // 8) * 8
                r = v * nl + j
                pltpu.async_copy(
                    table_hbm.at[pl.ds(ib, 8)], groups_v.at[pl.ds(r * 8, 8)], sem
                )

        def drain(j, carry):
            pltpu.make_async_copy(
                table_hbm.at[pl.ds(0, 8)], groups_v.at[pl.ds(0, 8)], sem
            ).wait()
            return carry

        lax.fori_loop(0, bpw, drain, 0)
        pltpu.sync_copy(groups_v, out_hbm.at[pl.ds(base * 8, bpw * 8)])

    return gather_kernel(table, idx)


def _tc_body(x_ref, g8_ref, r8_ref, o_ref):
    c = x_ref[...]
    g8 = g8_ref[...]
    r8 = r8_ref[...]

    bi = lax.broadcasted_iota(jnp.int32, (_NB, 8 * _NB), 0)
    mi = lax.broadcasted_iota(jnp.int32, (_NB, 8 * _NB), 1)
    sel = (mi == 8 * bi + r8).astype(jnp.bfloat16)
    gb = jax.lax.dot_general(
        sel, g8, (((1,), (0,)), ((), ())),
        preferred_element_type=jnp.float32,
    )

    s_all = jax.lax.dot_general(
        gb, c, (((1,), (0,)), ((), ())),
        preferred_element_type=jnp.float32,
    )
    row = lax.broadcasted_iota(jnp.int32, (_NB, _CB), 0)
    col = lax.broadcasted_iota(jnp.int32, (_NB, _CB), 1)
    d = col - row * _W
    mask = (d >= 0) & (d < _W)
    s = jnp.sum(jnp.where(mask, s_all, 0.0), axis=0)
    n2 = jnp.sum(c * c, axis=0)
    o_ref[...] = (s / jnp.maximum(jnp.sqrt(n2), 1e-12))[None, :]


def kernel(ctxt_word_vecs, ent_idxes, ent_embeddings):
    g8 = _sc_gather_groups(ent_embeddings.astype(jnp.bfloat16), ent_idxes)
    r8 = (ent_idxes % 8).astype(jnp.int32).reshape(_B, 1)
    xt = ctxt_word_vecs.T
    out = pl.pallas_call(
        _tc_body,
        grid=(_B * _W // _CB,),
        in_specs=[
            pl.BlockSpec((_D, _CB), lambda i: (0, i)),
            pl.BlockSpec((8 * _NB, _D), lambda i: (i, 0)),
            pl.BlockSpec((_NB, 1), lambda i: (i, 0)),
        ],
        out_specs=pl.BlockSpec((1, _CB), lambda i: (0, i)),
        out_shape=jax.ShapeDtypeStruct((1, _B * _W), jnp.float32),
    )(xt, g8, r8)
    return out.reshape(_B * 20, 5)


# speedup vs baseline: 3.3084x; 1.1167x over previous
import functools

import jax
import jax.numpy as jnp
from jax import lax
from jax.experimental import pallas as pl
from jax.experimental.pallas import tpu as pltpu
from jax.experimental.pallas import tpu_sc as plsc

_B = 1024
_W = 100
_D = 300
_V = 100000
_NB = 64
_CB = _NB * _W


def _sc_gather(table, idx):
    info = plsc.get_sparse_core_info()
    nc, ns, nl = info.num_cores, info.num_subcores, info.num_lanes
    nw = nc * ns
    bpw = _B // nw
    mesh = plsc.VectorSubcoreMesh(core_axis_name="c", subcore_axis_name="s")

    @functools.partial(
        pl.kernel,
        mesh=mesh,
        out_type=jax.ShapeDtypeStruct((_B, _D), jnp.float32),
        scratch_types=[
            pltpu.VMEM((bpw,), jnp.int32),
            pltpu.VMEM((bpw, _D), jnp.float32),
            pltpu.SemaphoreType.DMA,
        ],
        compiler_params=pltpu.CompilerParams(needs_layout_passes=False),
    )
    def gather_kernel(table_hbm, idx_hbm, out_hbm, idx_v, rows_v, sem):
        wid = lax.axis_index("s") * nc + lax.axis_index("c")
        base = wid * bpw
        pltpu.sync_copy(idx_hbm.at[pl.ds(base, bpw)], idx_v)

        lane = lax.iota(jnp.int32, nl)
        neg = jnp.full((nl,), -1, jnp.int32)
        for v in range(bpw // nl):
            vec = idx_v[pl.ds(v * nl, nl)]
            for j in range(nl):
                i = lax.reduce_max(jnp.where(lane == j, vec, neg), axes=(0,))
                pltpu.async_copy(table_hbm.at[i], rows_v.at[v * nl + j], sem)

        def drain(j, carry):
            pltpu.make_async_copy(table_hbm.at[0], rows_v.at[0], sem).wait()
            return carry

        lax.fori_loop(0, bpw, drain, 0)
        pltpu.sync_copy(rows_v, out_hbm.at[pl.ds(base, bpw)])

    return gather_kernel(table, idx)


def _tc_body(x_ref, g_ref, o_ref):
    c = x_ref[...]
    gb = g_ref[...]
    s_all = jax.lax.dot_general(
        gb, c, (((1,), (0,)), ((), ())),
        preferred_element_type=jnp.float32,
    )
    row = lax.broadcasted_iota(jnp.int32, (_NB, _CB), 0)
    col = lax.broadcasted_iota(jnp.int32, (_NB, _CB), 1)
    d = col - row * _W
    mask = (d >= 0) & (d < _W)
    s = jnp.sum(jnp.where(mask, s_all, 0.0), axis=0)
    n2 = jnp.sum(c * c, axis=0)
    o_ref[...] = (s / jnp.maximum(jnp.sqrt(n2), 1e-12))[None, :]


def kernel(ctxt_word_vecs, ent_idxes, ent_embeddings):
    g = _sc_gather(ent_embeddings, ent_idxes)
    xt = ctxt_word_vecs.T
    out = pl.pallas_call(
        _tc_body,
        grid=(_B * _W // _CB,),
        in_specs=[
            pl.BlockSpec((_D, _CB), lambda i: (0, i)),
            pl.BlockSpec((_NB, _D), lambda i: (i, 0)),
        ],
        out_specs=pl.BlockSpec((1, _CB), lambda i: (0, i)),
        out_shape=jax.ShapeDtypeStruct((1, _B * _W), jnp.float32),
    )(xt, g)
    return out.reshape(_B * 20, 5)
